# Initial kernel scaffold; baseline (speedup 1.0000x reference)
import jax
import jax.numpy as jnp
from jax.experimental import pallas as pl


def kernel(x, edge_index, W_conv, b_conv, W_gate, b_gate):
    raise NotImplementedError("write your pallas kernel here")


# speedup vs baseline: 4.6462x; 4.6462x over previous
import functools

import jax
import jax.numpy as jnp
from jax import lax
from jax.experimental import pallas as pl
from jax.experimental.pallas import tpu as pltpu
from jax.experimental.pallas import tpu_sc as plsc

N = 10000
E = 320000
D = 128

NC = 2
NS = 16
NW = NC * NS

CHUNK = 128
CPW = -(-E // (NW * CHUNK))
EPW = CPW * CHUNK
E_PAD = EPW * NW
N_PAD = 10240
ROWS_PT = N_PAD // NS

_mesh = plsc.VectorSubcoreMesh(core_axis_name="c", subcore_axis_name="s")


@functools.partial(
    pl.kernel,
    out_type=(
        jax.ShapeDtypeStruct((NC * N_PAD, D), jnp.float32),
        jax.ShapeDtypeStruct((NW * N_PAD,), jnp.float32),
    ),
    mesh=_mesh,
    scratch_types=(
        pltpu.VMEM_SHARED((N_PAD, D), jnp.float32),
        pltpu.VMEM((CHUNK,), jnp.int32),
        pltpu.VMEM((CHUNK,), jnp.int32),
        pltpu.VMEM((CHUNK, D), jnp.float32),
        pltpu.VMEM((N_PAD,), jnp.float32),
        pltpu.SemaphoreType.DMA,
    ),
    compiler_params=pltpu.CompilerParams(needs_layout_passes=False),
)
def _sc_aggregate(src_hbm, dst_hbm, x_hbm, sum_hbm, deg_hbm,
                  acc, sidx, didx, rows, degv, gsem):
    cid = lax.axis_index("c")
    sid = lax.axis_index("s")
    wid = cid * NS + sid

    zeros16 = jnp.zeros((16,), jnp.float32)

    def _zrows(i, carry):
        r = i // (D // 16)
        c = i % (D // 16)
        rows[r, pl.ds(c * 16, 16)] = zeros16
        return carry
    lax.fori_loop(0, CHUNK * (D // 16), _zrows, 0)
    for k in range(ROWS_PT // CHUNK):
        pltpu.sync_copy(rows, acc.at[pl.ds(sid * ROWS_PT + k * CHUNK, CHUNK)])

    def _zdeg(i, carry):
        degv[pl.ds(i * 16, 16)] = zeros16
        return carry
    lax.fori_loop(0, N_PAD // 16, _zdeg, 0)

    plsc.subcore_barrier()

    ones16 = jnp.ones((16,), jnp.float32)

    def _chunk(c, carry):
        off = wid * EPW + c * CHUNK
        pltpu.sync_copy(src_hbm.at[pl.ds(off, CHUNK)], sidx)
        pltpu.sync_copy(dst_hbm.at[pl.ds(off, CHUNK)], didx)
        pltpu.async_copy(x_hbm.at[sidx], rows, gsem).wait()
        pltpu.sync_copy(rows, acc.at[didx], add=True)
        for j in range(CHUNK // 16):
            dvec = didx[pl.ds(j * 16, 16)]
            plsc.addupdate_scatter(degv, [dvec], ones16)
        return carry

    lax.fori_loop(0, CPW, _chunk, 0)

    plsc.subcore_barrier()

    r0 = sid * ROWS_PT
    pltpu.sync_copy(acc.at[pl.ds(r0, ROWS_PT)],
                    sum_hbm.at[pl.ds(cid * N_PAD + r0, ROWS_PT)])
    pltpu.sync_copy(degv, deg_hbm.at[pl.ds(wid * N_PAD, N_PAD)])


BN = 1024


def _tc_body(x_ref, s_ref, dg_ref, wc_ref, bc_ref, wg_ref, bg_ref, o_ref):
    xb = x_ref[...]
    agg = s_ref[0] + s_ref[1]
    deg = jnp.maximum(jnp.sum(dg_ref[...], axis=0), 1.0)
    agg = agg / deg[:, None]
    conv = jnp.dot(agg, wc_ref[...], preferred_element_type=jnp.float32) + bc_ref[...]
    gate = jax.nn.sigmoid(
        jnp.dot(xb, wg_ref[...], preferred_element_type=jnp.float32) + bg_ref[...])
    o_ref[...] = gate * conv + xb - gate * xb


def _tc_gate(x, sums, degs, W_conv, b_conv, W_gate, b_gate):
    return pl.pallas_call(
        _tc_body,
        out_shape=jax.ShapeDtypeStruct((N_PAD, D), jnp.float32),
        grid=(N_PAD // BN,),
        in_specs=[
            pl.BlockSpec((BN, D), lambda i: (i, 0)),
            pl.BlockSpec((NC, BN, D), lambda i: (0, i, 0)),
            pl.BlockSpec((NW, BN), lambda i: (0, i)),
            pl.BlockSpec((D, D), lambda i: (0, 0)),
            pl.BlockSpec((1, D), lambda i: (0, 0)),
            pl.BlockSpec((D, D), lambda i: (0, 0)),
            pl.BlockSpec((1, D), lambda i: (0, 0)),
        ],
        out_specs=pl.BlockSpec((BN, D), lambda i: (i, 0)),
    )(x, sums, degs, W_conv, b_conv, W_gate, b_gate)


def kernel(x, edge_index, W_conv, b_conv, W_gate, b_gate):
    src = edge_index[0]
    dst = edge_index[1]
    pad = E_PAD - E
    src_p = jnp.concatenate([src, jnp.zeros((pad,), jnp.int32)])
    dst_p = jnp.concatenate([dst, jnp.full((pad,), N, jnp.int32)])
    sums, degs = _sc_aggregate(src_p, dst_p, x)
    sums = sums.reshape(NC, N_PAD, D)
    degs = degs.reshape(NW, N_PAD)
    out = _tc_gate(x, sums, degs, W_conv, b_conv.reshape(1, D),
                   W_gate, b_gate.reshape(1, D))
    return out[:N]
